# baseline (speedup 1.0000x reference)
import functools

import jax
import jax.numpy as jnp
from jax import lax
from jax.experimental import pallas as pl
from jax.experimental.pallas import tpu as pltpu
from jax.experimental.pallas import tpu_sc as plsc

N = 10000
E = 160000
D = 1280
H = 64
G = 64

NC = 2
NS = 16
HH = H // NC
E_PER_TILE = E // (NC * NS)
E_PER_TILE_C = E // NS
N_PER_TILE = N // NS


ROWS_BIG = 640
ROWS_LAST = N - 15 * ROWS_BIG


def _row_partition(s, copy_fn):
    @pl.when(s < NS - 1)
    def _():
        copy_fn(s * ROWS_BIG, ROWS_BIG)

    @pl.when(s == NS - 1)
    def _():
        copy_fn(15 * ROWS_BIG, ROWS_LAST)


DW = 16
CHUNKA = E_PER_TILE // 2


def _deg_body(ei_hbm, zeros_hbm, ones_hbm, deg_hbm, dst_v, ones_v, part_v,
              deg_sh, sem):
    c = lax.axis_index("c")
    s = lax.axis_index("s")

    def zinit(off, nr):
        pltpu.sync_copy(zeros_hbm.at[pl.ds(off, nr)], part_v.at[pl.ds(0, nr)])
        pltpu.sync_copy(part_v.at[pl.ds(0, nr)], deg_sh.at[pl.ds(off, nr)])

    _row_partition(s, zinit)
    plsc.subcore_barrier()
    w = c * NS + s
    dst_row = ei_hbm.at[1]
    pltpu.sync_copy(dst_row.at[pl.ds(w * E_PER_TILE, E_PER_TILE)], dst_v)
    pltpu.sync_copy(ones_hbm, ones_v)
    pltpu.async_copy(ones_v, deg_sh.at[dst_v], sem, add=True).wait()
    plsc.subcore_barrier()

    def wback(off, nr):
        pltpu.sync_copy(deg_sh.at[pl.ds(off, nr)], part_v.at[pl.ds(0, nr)])
        pltpu.sync_copy(part_v.at[pl.ds(0, nr)],
                        deg_hbm.at[c].at[pl.ds(off, nr)])

    _row_partition(s, wback)


def _deg_call(edge_index, zeros_n, ones_e):
    mesh = plsc.VectorSubcoreMesh(core_axis_name="c", subcore_axis_name="s",
                                  num_cores=NC, num_subcores=NS)
    return pl.kernel(
        _deg_body,
        out_type=jax.ShapeDtypeStruct((NC, N, DW), jnp.float32),
        mesh=mesh,
        scratch_types=[
            pltpu.VMEM((E_PER_TILE,), jnp.int32),
            pltpu.VMEM((E_PER_TILE, DW), jnp.float32),
            pltpu.VMEM((ROWS_BIG, DW), jnp.float32),
            pltpu.VMEM_SHARED((N, DW), jnp.float32),
            pltpu.SemaphoreType.DMA,
        ],
        compiler_params=pltpu.CompilerParams(use_tc_tiling_on_sc=False),
    )(edge_index, zeros_n, ones_e)


def _mm_body(nf_ref, w_ref, deg_ref, batch_ref, xs_ref, cnt_ref):
    i = pl.program_id(0)
    x = jnp.dot(nf_ref[...], w_ref[...], preferred_element_type=jnp.float32)
    deg = deg_ref[0][:, 0:1] + deg_ref[1][:, 0:1] + 1.0
    dinv = lax.rsqrt(deg)
    xs = x * dinv
    xs_ref[0] = xs[:, :HH]
    xs_ref[1] = xs[:, HH:]
    @pl.when(i == 0)
    def _():
        ids = lax.broadcasted_iota(jnp.int32, (G, N), 0)
        onehot = (batch_ref[...] == ids).astype(jnp.float32)
        part = jnp.sum(onehot, axis=1)[:, None]
        cnt_ref[...] = 1.0 / jnp.maximum(
            jnp.broadcast_to(part, (G, DW)), 1.0)


CHUNK = 400
NCHUNK = E_PER_TILE_C // CHUNK


def _rsqrt16(x):
    i = plsc.bitcast(x, jnp.int32)
    i = 0x5F3759DF - lax.shift_right_arithmetic(i, 1)
    y = plsc.bitcast(i, jnp.float32)
    for _ in range(3):
        y = y * (1.5 - 0.5 * x * y * y)
    return y


def _agg_body(xs_hbm, ei_hbm, degw_hbm, batch_hbm, b2_hbm, counts_hbm,
              pooled_hbm, sidx_v, didx_v, rows0, rows1, init_v,
              deg0_v, deg1_v, bat_big, bat_last, bias_v, cnt_v,
              pool_v, agg_sh, pooled_sh, gsem0, gsem1, ssem0, ssem1, psem):
    c = lax.axis_index("c")
    s = lax.axis_index("s")
    rows = [rows0, rows1]
    gsem = [gsem0, gsem1]
    ssem = [ssem0, ssem1]
    xs_c = xs_hbm.at[c]
    src_row = ei_hbm.at[0]
    dst_row = ei_hbm.at[1]

    for j in range(NCHUNK):
        off = s * E_PER_TILE_C + j * CHUNK
        pltpu.sync_copy(src_row.at[pl.ds(off, CHUNK)], sidx_v.at[j])
        pltpu.sync_copy(dst_row.at[pl.ds(off, CHUNK)], didx_v.at[j])

    @pl.when(s == 0)
    def _():
        for gi in range(G):
            pool_v[gi, pl.ds(0, 16)] = jnp.zeros((16,), jnp.float32)
            pool_v[gi, pl.ds(16, 16)] = jnp.zeros((16,), jnp.float32)
        pltpu.sync_copy(pool_v, pooled_sh)

    def xinit(off, nr):
        pltpu.sync_copy(xs_c.at[pl.ds(off, nr)], init_v.at[pl.ds(0, nr)])
        pltpu.sync_copy(init_v.at[pl.ds(0, nr)], agg_sh.at[pl.ds(off, nr)])

    _row_partition(s, xinit)

    g = [None, None]
    sc = [None, None]
    g[0] = pltpu.async_copy(xs_c.at[sidx_v.at[0]], rows[0], gsem[0])
    plsc.subcore_barrier()

    for j in range(NCHUNK):
        b = j & 1
        nb = (j + 1) & 1
        if j + 1 < NCHUNK:
            if j >= 1:
                sc[nb].wait()
            g[nb] = pltpu.async_copy(xs_c.at[sidx_v.at[j + 1]], rows[nb],
                                     gsem[nb])
        g[b].wait()
        sc[b] = pltpu.async_copy(rows[b], agg_sh.at[didx_v.at[j]], ssem[b],
                                 add=True)
    sc[(NCHUNK - 1) & 1].wait()
    sc[(NCHUNK - 2) & 1].wait()
    plsc.subcore_barrier()

    pltpu.sync_copy(b2_hbm.at[c], bias_v)
    bias0 = bias_v[0, pl.ds(0, 16)]
    bias1 = bias_v[0, pl.ds(16, 16)]

    def pool_rows(off, nr, bat_v):
        pltpu.sync_copy(agg_sh.at[pl.ds(off, nr)], init_v.at[pl.ds(0, nr)])
        pltpu.sync_copy(degw_hbm.at[0].at[pl.ds(off, nr)],
                        deg0_v.at[pl.ds(0, nr)])
        pltpu.sync_copy(degw_hbm.at[1].at[pl.ds(off, nr)],
                        deg1_v.at[pl.ds(0, nr)])
        pltpu.sync_copy(batch_hbm.at[pl.ds(off, nr)], bat_v)

        def row_fn(i, carry):
            dvec = deg0_v[i, pl.ds(0, 16)] + deg1_v[i, pl.ds(0, 16)] + 1.0
            r = _rsqrt16(dvec)
            p0 = jnp.maximum(init_v[i, pl.ds(0, 16)] * r + bias0, 0.0)
            p1 = jnp.maximum(init_v[i, pl.ds(16, 16)] * r + bias1, 0.0)
            init_v[i, pl.ds(0, 16)] = p0
            init_v[i, pl.ds(16, 16)] = p1
            return carry

        lax.fori_loop(0, nr, row_fn, 0)
        pltpu.async_copy(init_v.at[pl.ds(0, nr)], pooled_sh.at[bat_v],
                         psem, add=True).wait()

    @pl.when(s < NS - 1)
    def _():
        pool_rows(s * ROWS_BIG, ROWS_BIG, bat_big)

    @pl.when(s == NS - 1)
    def _():
        pool_rows(15 * ROWS_BIG, ROWS_LAST, bat_last)

    plsc.subcore_barrier()

    @pl.when(s == 0)
    def _():
        pltpu.sync_copy(pooled_sh, pool_v)
        pltpu.sync_copy(counts_hbm, cnt_v)

        def div_fn(gi, carry):
            cv = cnt_v[gi, pl.ds(0, 16)]
            pool_v[gi, pl.ds(0, 16)] = pool_v[gi, pl.ds(0, 16)] * cv
            pool_v[gi, pl.ds(16, 16)] = pool_v[gi, pl.ds(16, 16)] * cv
            return carry

        lax.fori_loop(0, G, div_fn, 0)
        pltpu.sync_copy(pool_v, pooled_hbm.at[c])


def _agg_call(xs, edge_index, deg_wide, batch, b2, counts):
    mesh = plsc.VectorSubcoreMesh(core_axis_name="c", subcore_axis_name="s",
                                  num_cores=NC, num_subcores=NS)
    return pl.kernel(
        _agg_body,
        out_type=jax.ShapeDtypeStruct((NC, G, HH), jnp.float32),
        mesh=mesh,
        scratch_types=[
            pltpu.VMEM((NCHUNK, CHUNK), jnp.int32),
            pltpu.VMEM((NCHUNK, CHUNK), jnp.int32),
            pltpu.VMEM((CHUNK, HH), jnp.float32),
            pltpu.VMEM((CHUNK, HH), jnp.float32),
            pltpu.VMEM((ROWS_BIG, HH), jnp.float32),
            pltpu.VMEM((ROWS_BIG, DW), jnp.float32),
            pltpu.VMEM((ROWS_BIG, DW), jnp.float32),
            pltpu.VMEM((ROWS_BIG,), jnp.int32),
            pltpu.VMEM((ROWS_LAST,), jnp.int32),
            pltpu.VMEM((1, HH), jnp.float32),
            pltpu.VMEM((G, DW), jnp.float32),
            pltpu.VMEM((G, HH), jnp.float32),
            pltpu.VMEM_SHARED((N, HH), jnp.float32),
            pltpu.VMEM_SHARED((G, HH), jnp.float32),
            pltpu.SemaphoreType.DMA,
            pltpu.SemaphoreType.DMA,
            pltpu.SemaphoreType.DMA,
            pltpu.SemaphoreType.DMA,
            pltpu.SemaphoreType.DMA,
        ],
        compiler_params=pltpu.CompilerParams(use_tc_tiling_on_sc=False,
                                             needs_layout_passes=False),
    )(xs, edge_index, deg_wide, batch, b2, counts)


MM_BLK = 2000


@jax.jit
def kernel(node_feat, edge_index, batch, W, b):
    zeros_n = jnp.zeros((N, DW), jnp.float32)
    ones_e = jnp.ones((E_PER_TILE, DW), jnp.float32)
    deg_wide = _deg_call(edge_index, zeros_n, ones_e)
    xs, counts = pl.pallas_call(
        _mm_body,
        grid=(N // MM_BLK,),
        in_specs=[
            pl.BlockSpec((MM_BLK, D), lambda i: (i, 0)),
            pl.BlockSpec((D, H), lambda i: (0, 0)),
            pl.BlockSpec((NC, MM_BLK, DW), lambda i: (0, i, 0)),
            pl.BlockSpec((1, N), lambda i: (0, 0)),
        ],
        out_specs=[
            pl.BlockSpec((NC, MM_BLK, HH), lambda i: (0, i, 0)),
            pl.BlockSpec((G, DW), lambda i: (0, 0)),
        ],
        out_shape=[
            jax.ShapeDtypeStruct((NC, N, HH), jnp.float32),
            jax.ShapeDtypeStruct((G, DW), jnp.float32),
        ],
    )(node_feat, W, deg_wide, batch.reshape(1, N))
    pooled2 = _agg_call(xs, edge_index, deg_wide, batch,
                        b.reshape(NC, 1, HH), counts)
    return jnp.concatenate([pooled2[0], pooled2[1]], axis=1)


# speedup vs baseline: 1.1093x; 1.1093x over previous
import functools

import jax
import jax.numpy as jnp
from jax import lax
from jax.experimental import pallas as pl
from jax.experimental.pallas import tpu as pltpu
from jax.experimental.pallas import tpu_sc as plsc

N = 10000
E = 160000
D = 1280
H = 64
G = 64

NC = 2
NS = 16
HH = H // NC
E_PER_TILE = E // (NC * NS)
E_PER_TILE_C = E // NS
N_PER_TILE = N // NS


ROWS_BIG = 640
ROWS_LAST = N - 15 * ROWS_BIG


def _row_partition(s, copy_fn):
    @pl.when(s < NS - 1)
    def _():
        copy_fn(s * ROWS_BIG, ROWS_BIG)

    @pl.when(s == NS - 1)
    def _():
        copy_fn(15 * ROWS_BIG, ROWS_LAST)


DW = 16
CHUNKA = E_PER_TILE // 2


def _deg_body(ei_hbm, zeros_hbm, ones_hbm, deg_hbm, dst_v, ones_v, part_v,
              deg_sh, sem):
    c = lax.axis_index("c")
    s = lax.axis_index("s")

    def zinit(off, nr):
        pltpu.sync_copy(zeros_hbm.at[pl.ds(off, nr)], part_v.at[pl.ds(0, nr)])
        pltpu.sync_copy(part_v.at[pl.ds(0, nr)], deg_sh.at[pl.ds(off, nr)])

    _row_partition(s, zinit)
    plsc.subcore_barrier()
    w = c * NS + s
    dst_row = ei_hbm.at[1]
    pltpu.sync_copy(dst_row.at[pl.ds(w * E_PER_TILE, E_PER_TILE)], dst_v)
    pltpu.sync_copy(ones_hbm, ones_v)
    pltpu.async_copy(ones_v, deg_sh.at[dst_v], sem, add=True).wait()
    plsc.subcore_barrier()

    def wback(off, nr):
        pltpu.sync_copy(deg_sh.at[pl.ds(off, nr)], part_v.at[pl.ds(0, nr)])
        pltpu.sync_copy(part_v.at[pl.ds(0, nr)],
                        deg_hbm.at[c].at[pl.ds(off, nr)])

    _row_partition(s, wback)


def _deg_call(edge_index, zeros_n, ones_e):
    mesh = plsc.VectorSubcoreMesh(core_axis_name="c", subcore_axis_name="s",
                                  num_cores=NC, num_subcores=NS)
    return pl.kernel(
        _deg_body,
        out_type=jax.ShapeDtypeStruct((NC, N, DW), jnp.float32),
        mesh=mesh,
        scratch_types=[
            pltpu.VMEM((E_PER_TILE,), jnp.int32),
            pltpu.VMEM((E_PER_TILE, DW), jnp.float32),
            pltpu.VMEM((ROWS_BIG, DW), jnp.float32),
            pltpu.VMEM_SHARED((N, DW), jnp.float32),
            pltpu.SemaphoreType.DMA,
        ],
        compiler_params=pltpu.CompilerParams(use_tc_tiling_on_sc=False),
    )(edge_index, zeros_n, ones_e)


def _mm_body(nf_ref, w_ref, deg_ref, xs_ref, dinv_ref):
    x = jnp.dot(nf_ref[...], w_ref[...], preferred_element_type=jnp.float32)
    deg = deg_ref[0][:, 0:1] + deg_ref[1][:, 0:1] + 1.0
    dinv = lax.rsqrt(deg)
    xs = x * dinv
    dinv_ref[...] = dinv
    xs_ref[0] = xs[:, :HH]
    xs_ref[1] = xs[:, HH:]


CHUNK = 1000
NCHUNK = E_PER_TILE_C // CHUNK


def _agg_body(xs_hbm, ei_hbm, agg_hbm, sidx_v, didx_v,
              rows0, rows1, init_v, agg_sh, gsem0, gsem1, ssem0, ssem1):
    c = lax.axis_index("c")
    s = lax.axis_index("s")
    rows = [rows0, rows1]
    gsem = [gsem0, gsem1]
    ssem = [ssem0, ssem1]
    xs_c = xs_hbm.at[c]
    src_row = ei_hbm.at[0]
    dst_row = ei_hbm.at[1]

    for j in range(NCHUNK):
        off = s * E_PER_TILE_C + j * CHUNK
        pltpu.sync_copy(src_row.at[pl.ds(off, CHUNK)], sidx_v.at[j])
        pltpu.sync_copy(dst_row.at[pl.ds(off, CHUNK)], didx_v.at[j])

    def xinit(off, nr):
        pltpu.sync_copy(xs_c.at[pl.ds(off, nr)], init_v.at[pl.ds(0, nr)])
        pltpu.sync_copy(init_v.at[pl.ds(0, nr)], agg_sh.at[pl.ds(off, nr)])

    _row_partition(s, xinit)

    g = [None, None]
    sc = [None, None]
    g[0] = pltpu.async_copy(xs_c.at[sidx_v.at[0]], rows[0], gsem[0])
    plsc.subcore_barrier()

    for j in range(NCHUNK):
        b = j & 1
        nb = (j + 1) & 1
        if j + 1 < NCHUNK:
            if j >= 1:
                sc[nb].wait()
            g[nb] = pltpu.async_copy(xs_c.at[sidx_v.at[j + 1]], rows[nb],
                                     gsem[nb])
        g[b].wait()
        sc[b] = pltpu.async_copy(rows[b], agg_sh.at[didx_v.at[j]], ssem[b],
                                 add=True)
    sc[(NCHUNK - 1) & 1].wait()
    sc[(NCHUNK - 2) & 1].wait()
    plsc.subcore_barrier()

    def wback(off, nr):
        pltpu.sync_copy(agg_sh.at[pl.ds(off, nr)], init_v.at[pl.ds(0, nr)])
        pltpu.sync_copy(init_v.at[pl.ds(0, nr)],
                        agg_hbm.at[c].at[pl.ds(off, nr)])

    _row_partition(s, wback)


def _agg_call(xs, edge_index):
    mesh = plsc.VectorSubcoreMesh(core_axis_name="c", subcore_axis_name="s",
                                  num_cores=NC, num_subcores=NS)
    return pl.kernel(
        _agg_body,
        out_type=jax.ShapeDtypeStruct((NC, N, HH), jnp.float32),
        mesh=mesh,
        scratch_types=[
            pltpu.VMEM((NCHUNK, CHUNK), jnp.int32),
            pltpu.VMEM((NCHUNK, CHUNK), jnp.int32),
            pltpu.VMEM((CHUNK, HH), jnp.float32),
            pltpu.VMEM((CHUNK, HH), jnp.float32),
            pltpu.VMEM((ROWS_BIG, HH), jnp.float32),
            pltpu.VMEM_SHARED((N, HH), jnp.float32),
            pltpu.SemaphoreType.DMA,
            pltpu.SemaphoreType.DMA,
            pltpu.SemaphoreType.DMA,
            pltpu.SemaphoreType.DMA,
        ],
        compiler_params=pltpu.CompilerParams(use_tc_tiling_on_sc=False),
    )(xs, edge_index)


def _pool_body(agg_ref, dinv_ref, b_ref, batch_ref, out_ref):
    agg = jnp.concatenate([agg_ref[0], agg_ref[1]], axis=1)
    dinv = dinv_ref[...]
    p = jnp.maximum(agg * dinv + b_ref[...], 0.0)
    ids = lax.broadcasted_iota(jnp.int32, (N, G), 1)
    onehot = (batch_ref[...] == ids).astype(jnp.float32)
    sums = lax.dot_general(onehot, p, (((0,), (0,)), ((), ())),
                           preferred_element_type=jnp.float32)
    counts = jnp.sum(onehot, axis=0)[:, None]
    out_ref[...] = sums / jnp.maximum(counts, 1.0)


def _pool_call(agg, dinv, b, batch):
    return pl.pallas_call(
        _pool_body,
        out_shape=jax.ShapeDtypeStruct((G, H), jnp.float32),
    )(agg, dinv, b, batch)


MM_BLK = 1000


@jax.jit
def kernel(node_feat, edge_index, batch, W, b):
    zeros_n = jnp.zeros((N, DW), jnp.float32)
    ones_e = jnp.ones((E_PER_TILE, DW), jnp.float32)
    deg_wide = _deg_call(edge_index, zeros_n, ones_e)
    xs, dinv = pl.pallas_call(
        _mm_body,
        grid=(N // MM_BLK,),
        in_specs=[
            pl.BlockSpec((MM_BLK, D), lambda i: (i, 0)),
            pl.BlockSpec((D, H), lambda i: (0, 0)),
            pl.BlockSpec((NC, MM_BLK, DW), lambda i: (0, i, 0)),
        ],
        out_specs=[
            pl.BlockSpec((NC, MM_BLK, HH), lambda i: (0, i, 0)),
            pl.BlockSpec((MM_BLK, 1), lambda i: (i, 0)),
        ],
        out_shape=[
            jax.ShapeDtypeStruct((NC, N, HH), jnp.float32),
            jax.ShapeDtypeStruct((N, 1), jnp.float32),
        ],
    )(node_feat, W, deg_wide)
    agg = _agg_call(xs, edge_index)
    pooled = _pool_call(agg, dinv, b.reshape(1, H), batch.reshape(N, 1))
    return pooled


# speedup vs baseline: 1.1347x; 1.0229x over previous
import functools

import jax
import jax.numpy as jnp
from jax import lax
from jax.experimental import pallas as pl
from jax.experimental.pallas import tpu as pltpu
from jax.experimental.pallas import tpu_sc as plsc

N = 10000
E = 160000
D = 1280
H = 64
G = 64

NC = 2
NS = 16
HH = H // NC
E_PER_TILE = E // (NC * NS)
E_PER_TILE_C = E // NS
N_PER_TILE = N // NS


ROWS_BIG = 640
ROWS_LAST = N - 15 * ROWS_BIG


def _row_partition(s, copy_fn):
    @pl.when(s < NS - 1)
    def _():
        copy_fn(s * ROWS_BIG, ROWS_BIG)

    @pl.when(s == NS - 1)
    def _():
        copy_fn(15 * ROWS_BIG, ROWS_LAST)


DW = 16
CHUNKA = E_PER_TILE // 2


def _deg_body(dst_hbm, zeros_hbm, ones_hbm, deg_hbm, dst_v, ones_v, part_v,
              deg_sh, sem):
    c = lax.axis_index("c")
    s = lax.axis_index("s")

    def zinit(off, nr):
        pltpu.sync_copy(zeros_hbm.at[pl.ds(off, nr)], part_v.at[pl.ds(0, nr)])
        pltpu.sync_copy(part_v.at[pl.ds(0, nr)], deg_sh.at[pl.ds(off, nr)])

    _row_partition(s, zinit)
    plsc.subcore_barrier()
    w = c * NS + s
    pltpu.sync_copy(dst_hbm.at[pl.ds(w * E_PER_TILE, E_PER_TILE)], dst_v)
    pltpu.sync_copy(ones_hbm, ones_v)
    pltpu.async_copy(ones_v, deg_sh.at[dst_v], sem, add=True).wait()
    plsc.subcore_barrier()

    def wback(off, nr):
        pltpu.sync_copy(deg_sh.at[pl.ds(off, nr)], part_v.at[pl.ds(0, nr)])
        pltpu.sync_copy(part_v.at[pl.ds(0, nr)],
                        deg_hbm.at[c].at[pl.ds(off, nr)])

    _row_partition(s, wback)


def _deg_call(dst, zeros_n, ones_e):
    mesh = plsc.VectorSubcoreMesh(core_axis_name="c", subcore_axis_name="s",
                                  num_cores=NC, num_subcores=NS)
    return pl.kernel(
        _deg_body,
        out_type=jax.ShapeDtypeStruct((NC, N, DW), jnp.float32),
        mesh=mesh,
        scratch_types=[
            pltpu.VMEM((E_PER_TILE,), jnp.int32),
            pltpu.VMEM((E_PER_TILE, DW), jnp.float32),
            pltpu.VMEM((ROWS_BIG, DW), jnp.float32),
            pltpu.VMEM_SHARED((N, DW), jnp.float32),
            pltpu.SemaphoreType.DMA,
        ],
        compiler_params=pltpu.CompilerParams(use_tc_tiling_on_sc=False),
    )(dst, zeros_n, ones_e)


def _mm_body(nf_ref, w_ref, deg_ref, xs_ref, dinv_ref):
    x = jnp.dot(nf_ref[...], w_ref[...], preferred_element_type=jnp.float32)
    deg = deg_ref[0][:, 0:1] + deg_ref[1][:, 0:1] + 1.0
    dinv = lax.rsqrt(deg)
    xs = x * dinv
    dinv_ref[...] = dinv
    xs_ref[0] = xs[:, :HH]
    xs_ref[1] = xs[:, HH:]


CHUNK = 1000
NCHUNK = E_PER_TILE_C // CHUNK


def _agg_body(xs_hbm, src_hbm, dst_hbm, agg_hbm, sidx_v, didx_v,
              rows0, rows1, init_v, agg_sh, gsem0, gsem1, ssem0, ssem1):
    c = lax.axis_index("c")
    s = lax.axis_index("s")
    rows = [rows0, rows1]
    gsem = [gsem0, gsem1]
    ssem = [ssem0, ssem1]
    xs_c = xs_hbm.at[c]

    pltpu.sync_copy(src_hbm.at[pl.ds(s * NCHUNK, NCHUNK)], sidx_v)
    pltpu.sync_copy(dst_hbm.at[pl.ds(s * NCHUNK, NCHUNK)], didx_v)

    def xinit(off, nr):
        pltpu.sync_copy(xs_c.at[pl.ds(off, nr)], init_v.at[pl.ds(0, nr)])
        pltpu.sync_copy(init_v.at[pl.ds(0, nr)], agg_sh.at[pl.ds(off, nr)])

    _row_partition(s, xinit)

    g = [None, None]
    sc = [None, None]
    g[0] = pltpu.async_copy(xs_c.at[sidx_v.at[0]], rows[0], gsem[0])
    plsc.subcore_barrier()

    for j in range(NCHUNK):
        b = j & 1
        nb = (j + 1) & 1
        if j + 1 < NCHUNK:
            if j >= 1:
                sc[nb].wait()
            g[nb] = pltpu.async_copy(xs_c.at[sidx_v.at[j + 1]], rows[nb],
                                     gsem[nb])
        g[b].wait()
        sc[b] = pltpu.async_copy(rows[b], agg_sh.at[didx_v.at[j]], ssem[b],
                                 add=True)
    sc[(NCHUNK - 1) & 1].wait()
    sc[(NCHUNK - 2) & 1].wait()
    plsc.subcore_barrier()

    def wback(off, nr):
        pltpu.sync_copy(agg_sh.at[pl.ds(off, nr)], init_v.at[pl.ds(0, nr)])
        pltpu.sync_copy(init_v.at[pl.ds(0, nr)],
                        agg_hbm.at[c].at[pl.ds(off, nr)])

    _row_partition(s, wback)


def _agg_call(xs, src2, dst2):
    mesh = plsc.VectorSubcoreMesh(core_axis_name="c", subcore_axis_name="s",
                                  num_cores=NC, num_subcores=NS)
    return pl.kernel(
        _agg_body,
        out_type=jax.ShapeDtypeStruct((NC, N, HH), jnp.float32),
        mesh=mesh,
        scratch_types=[
            pltpu.VMEM((NCHUNK, CHUNK), jnp.int32),
            pltpu.VMEM((NCHUNK, CHUNK), jnp.int32),
            pltpu.VMEM((CHUNK, HH), jnp.float32),
            pltpu.VMEM((CHUNK, HH), jnp.float32),
            pltpu.VMEM((ROWS_BIG, HH), jnp.float32),
            pltpu.VMEM_SHARED((N, HH), jnp.float32),
            pltpu.SemaphoreType.DMA,
            pltpu.SemaphoreType.DMA,
            pltpu.SemaphoreType.DMA,
            pltpu.SemaphoreType.DMA,
        ],
        compiler_params=pltpu.CompilerParams(use_tc_tiling_on_sc=False),
    )(xs, src2, dst2)


def _pool_body(agg_ref, dinv_ref, b_ref, batch_ref, out_ref):
    agg = jnp.concatenate([agg_ref[0], agg_ref[1]], axis=1)
    dinv = dinv_ref[...]
    p = jnp.maximum(agg * dinv + b_ref[...], 0.0)
    ids = lax.broadcasted_iota(jnp.int32, (N, G), 1)
    onehot = (batch_ref[...] == ids).astype(jnp.float32)
    sums = lax.dot_general(onehot, p, (((0,), (0,)), ((), ())),
                           preferred_element_type=jnp.float32)
    counts = jnp.sum(onehot, axis=0)[:, None]
    out_ref[...] = sums / jnp.maximum(counts, 1.0)


def _pool_call(agg, dinv, b, batch):
    return pl.pallas_call(
        _pool_body,
        out_shape=jax.ShapeDtypeStruct((G, H), jnp.float32),
    )(agg, dinv, b, batch)


MM_BLK = 1000


@jax.jit
def kernel(node_feat, edge_index, batch, W, b):
    src = edge_index[0]
    dst = edge_index[1]
    zeros_n = jnp.zeros((N, DW), jnp.float32)
    ones_e = jnp.ones((E_PER_TILE, DW), jnp.float32)
    deg_wide = _deg_call(dst, zeros_n, ones_e)
    xs, dinv = pl.pallas_call(
        _mm_body,
        grid=(N // MM_BLK,),
        in_specs=[
            pl.BlockSpec((MM_BLK, D), lambda i: (i, 0)),
            pl.BlockSpec((D, H), lambda i: (0, 0)),
            pl.BlockSpec((NC, MM_BLK, DW), lambda i: (0, i, 0)),
        ],
        out_specs=[
            pl.BlockSpec((NC, MM_BLK, HH), lambda i: (0, i, 0)),
            pl.BlockSpec((MM_BLK, 1), lambda i: (i, 0)),
        ],
        out_shape=[
            jax.ShapeDtypeStruct((NC, N, HH), jnp.float32),
            jax.ShapeDtypeStruct((N, 1), jnp.float32),
        ],
    )(node_feat, W, deg_wide)
    agg = _agg_call(xs, src.reshape(NS * NCHUNK, CHUNK),
                    dst.reshape(NS * NCHUNK, CHUNK))
    pooled = _pool_call(agg, dinv, b.reshape(1, H), batch.reshape(N, 1))
    return pooled


# speedup vs baseline: 1.1369x; 1.0020x over previous
import functools

import jax
import jax.numpy as jnp
from jax import lax
from jax.experimental import pallas as pl
from jax.experimental.pallas import tpu as pltpu
from jax.experimental.pallas import tpu_sc as plsc

N = 10000
E = 160000
D = 1280
H = 64
G = 64

NC = 2
NS = 16
HH = H // NC
E_PER_TILE = E // (NC * NS)
E_PER_TILE_C = E // NS
N_PER_TILE = N // NS


ROWS_BIG = 640
ROWS_LAST = N - 15 * ROWS_BIG


def _row_partition(s, copy_fn):
    @pl.when(s < NS - 1)
    def _():
        copy_fn(s * ROWS_BIG, ROWS_BIG)

    @pl.when(s == NS - 1)
    def _():
        copy_fn(15 * ROWS_BIG, ROWS_LAST)


DW = 16
CHUNKA = E_PER_TILE // 2


def _deg_body(dst_hbm, zeros_hbm, ones_hbm, deg_hbm, dst_v, ones_v, part_v,
              deg_sh, sem):
    c = lax.axis_index("c")
    s = lax.axis_index("s")

    def zinit(off, nr):
        pltpu.sync_copy(zeros_hbm.at[pl.ds(off, nr)], part_v.at[pl.ds(0, nr)])
        pltpu.sync_copy(part_v.at[pl.ds(0, nr)], deg_sh.at[pl.ds(off, nr)])

    _row_partition(s, zinit)
    plsc.subcore_barrier()
    w = c * NS + s
    pltpu.sync_copy(dst_hbm.at[pl.ds(w * E_PER_TILE, E_PER_TILE)], dst_v)
    pltpu.sync_copy(ones_hbm, ones_v)
    pltpu.async_copy(ones_v, deg_sh.at[dst_v], sem, add=True).wait()
    plsc.subcore_barrier()

    def wback(off, nr):
        pltpu.sync_copy(deg_sh.at[pl.ds(off, nr)], part_v.at[pl.ds(0, nr)])
        pltpu.sync_copy(part_v.at[pl.ds(0, nr)],
                        deg_hbm.at[c].at[pl.ds(off, nr)])

    _row_partition(s, wback)


def _deg_call(dst, zeros_n, ones_e):
    mesh = plsc.VectorSubcoreMesh(core_axis_name="c", subcore_axis_name="s",
                                  num_cores=NC, num_subcores=NS)
    return pl.kernel(
        _deg_body,
        out_type=jax.ShapeDtypeStruct((NC, N, DW), jnp.float32),
        mesh=mesh,
        scratch_types=[
            pltpu.VMEM((E_PER_TILE,), jnp.int32),
            pltpu.VMEM((E_PER_TILE, DW), jnp.float32),
            pltpu.VMEM((ROWS_BIG, DW), jnp.float32),
            pltpu.VMEM_SHARED((N, DW), jnp.float32),
            pltpu.SemaphoreType.DMA,
        ],
        compiler_params=pltpu.CompilerParams(use_tc_tiling_on_sc=False),
    )(dst, zeros_n, ones_e)


def _mm_body(nf_ref, w_ref, deg_ref, xs_ref, dinv_ref):
    x = jnp.dot(nf_ref[...], w_ref[...], preferred_element_type=jnp.float32)
    deg = deg_ref[0] + deg_ref[1] + 1.0
    dinv = lax.rsqrt(deg)
    xs = x * dinv
    dinv_ref[...] = dinv
    xs_ref[0] = xs[:, :HH]
    xs_ref[1] = xs[:, HH:]


CHUNK = 1000
NCHUNK = E_PER_TILE_C // CHUNK


def _agg_body(xs_hbm, src_hbm, dst_hbm, agg_hbm, sidx_v, didx_v,
              rows0, rows1, init_v, agg_sh, gsem0, gsem1, ssem0, ssem1):
    c = lax.axis_index("c")
    s = lax.axis_index("s")
    rows = [rows0, rows1]
    gsem = [gsem0, gsem1]
    ssem = [ssem0, ssem1]
    xs_c = xs_hbm.at[c]

    pltpu.sync_copy(src_hbm.at[pl.ds(s * NCHUNK, NCHUNK)], sidx_v)
    pltpu.sync_copy(dst_hbm.at[pl.ds(s * NCHUNK, NCHUNK)], didx_v)

    def xinit(off, nr):
        pltpu.sync_copy(xs_c.at[pl.ds(off, nr)], init_v.at[pl.ds(0, nr)])
        pltpu.sync_copy(init_v.at[pl.ds(0, nr)], agg_sh.at[pl.ds(off, nr)])

    _row_partition(s, xinit)

    g = [None, None]
    sc = [None, None]
    g[0] = pltpu.async_copy(xs_c.at[sidx_v.at[0]], rows[0], gsem[0])
    plsc.subcore_barrier()

    for j in range(NCHUNK):
        b = j & 1
        nb = (j + 1) & 1
        if j + 1 < NCHUNK:
            if j >= 1:
                sc[nb].wait()
            g[nb] = pltpu.async_copy(xs_c.at[sidx_v.at[j + 1]], rows[nb],
                                     gsem[nb])
        g[b].wait()
        sc[b] = pltpu.async_copy(rows[b], agg_sh.at[didx_v.at[j]], ssem[b],
                                 add=True)
    sc[(NCHUNK - 1) & 1].wait()
    sc[(NCHUNK - 2) & 1].wait()
    plsc.subcore_barrier()

    def wback(off, nr):
        pltpu.sync_copy(agg_sh.at[pl.ds(off, nr)], init_v.at[pl.ds(0, nr)])
        pltpu.sync_copy(init_v.at[pl.ds(0, nr)],
                        agg_hbm.at[c].at[pl.ds(off, nr)])

    _row_partition(s, wback)


def _agg_call(xs, src2, dst2):
    mesh = plsc.VectorSubcoreMesh(core_axis_name="c", subcore_axis_name="s",
                                  num_cores=NC, num_subcores=NS)
    return pl.kernel(
        _agg_body,
        out_type=jax.ShapeDtypeStruct((NC, N, HH), jnp.float32),
        mesh=mesh,
        scratch_types=[
            pltpu.VMEM((NCHUNK, CHUNK), jnp.int32),
            pltpu.VMEM((NCHUNK, CHUNK), jnp.int32),
            pltpu.VMEM((CHUNK, HH), jnp.float32),
            pltpu.VMEM((CHUNK, HH), jnp.float32),
            pltpu.VMEM((ROWS_BIG, HH), jnp.float32),
            pltpu.VMEM_SHARED((N, HH), jnp.float32),
            pltpu.SemaphoreType.DMA,
            pltpu.SemaphoreType.DMA,
            pltpu.SemaphoreType.DMA,
            pltpu.SemaphoreType.DMA,
        ],
        compiler_params=pltpu.CompilerParams(use_tc_tiling_on_sc=False),
    )(xs, src2, dst2)


def _pool_body(agg_ref, dinv_ref, b_ref, batch_ref, out_ref):
    agg = jnp.concatenate([agg_ref[0], agg_ref[1]], axis=1)
    dinv = dinv_ref[...]
    p = jnp.maximum(agg * dinv + b_ref[...], 0.0)
    ids = lax.broadcasted_iota(jnp.int32, (N, G), 1)
    onehot = (batch_ref[...] == ids).astype(jnp.float32)
    sums = lax.dot_general(onehot, p, (((0,), (0,)), ((), ())),
                           preferred_element_type=jnp.float32)
    counts = jnp.sum(onehot, axis=0)[:, None]
    out_ref[...] = sums / jnp.maximum(counts, 1.0)


def _pool_call(agg, dinv, b, batch):
    return pl.pallas_call(
        _pool_body,
        out_shape=jax.ShapeDtypeStruct((G, H), jnp.float32),
    )(agg, dinv, b, batch)


MM_BLK = 1000


@jax.jit
def kernel(node_feat, edge_index, batch, W, b):
    src = edge_index[0]
    dst = edge_index[1]
    zeros_n = jnp.zeros((N, DW), jnp.float32)
    ones_e = jnp.ones((E_PER_TILE, DW), jnp.float32)
    deg_wide = _deg_call(dst, zeros_n, ones_e)
    deg_part = deg_wide[:, :, 0:1]
    xs, dinv = pl.pallas_call(
        _mm_body,
        grid=(N // MM_BLK,),
        in_specs=[
            pl.BlockSpec((MM_BLK, D), lambda i: (i, 0)),
            pl.BlockSpec((D, H), lambda i: (0, 0)),
            pl.BlockSpec((NC, MM_BLK, 1), lambda i: (0, i, 0)),
        ],
        out_specs=[
            pl.BlockSpec((NC, MM_BLK, HH), lambda i: (0, i, 0)),
            pl.BlockSpec((MM_BLK, 1), lambda i: (i, 0)),
        ],
        out_shape=[
            jax.ShapeDtypeStruct((NC, N, HH), jnp.float32),
            jax.ShapeDtypeStruct((N, 1), jnp.float32),
        ],
    )(node_feat, W, deg_part)
    agg = _agg_call(xs, src.reshape(NS * NCHUNK, CHUNK),
                    dst.reshape(NS * NCHUNK, CHUNK))
    pooled = _pool_call(agg, dinv, b.reshape(1, H), batch.reshape(N, 1))
    return pooled


# speedup vs baseline: 1.1631x; 1.0230x over previous
import functools

import jax
import jax.numpy as jnp
from jax import lax
from jax.experimental import pallas as pl
from jax.experimental.pallas import tpu as pltpu
from jax.experimental.pallas import tpu_sc as plsc

N = 10000
E = 160000
D = 1280
H = 64
G = 64

NC = 2
NS = 16
HH = H // NC
E_PER_TILE = E // (NC * NS)
E_PER_TILE_C = E // NS
N_PER_TILE = N // NS


ROWS_BIG = 640
ROWS_LAST = N - 15 * ROWS_BIG


def _row_partition(s, copy_fn):
    @pl.when(s < NS - 1)
    def _():
        copy_fn(s * ROWS_BIG, ROWS_BIG)

    @pl.when(s == NS - 1)
    def _():
        copy_fn(15 * ROWS_BIG, ROWS_LAST)


DW = 16
CHUNKA = E_PER_TILE // 2


def _deg_body(dst_hbm, zeros_hbm, ones_hbm, deg_hbm, dst_v, ones_v, part_v,
              deg_sh, sem):
    c = lax.axis_index("c")
    s = lax.axis_index("s")

    def zinit(off, nr):
        pltpu.sync_copy(zeros_hbm.at[pl.ds(off, nr)], part_v.at[pl.ds(0, nr)])
        pltpu.sync_copy(part_v.at[pl.ds(0, nr)], deg_sh.at[pl.ds(off, nr)])

    _row_partition(s, zinit)
    plsc.subcore_barrier()
    w = c * NS + s
    pltpu.sync_copy(dst_hbm.at[pl.ds(2 * w, 2)], dst_v)
    pltpu.sync_copy(ones_hbm, ones_v)
    pltpu.async_copy(ones_v, deg_sh.at[dst_v.at[0]], sem, add=True).wait()
    pltpu.async_copy(ones_v, deg_sh.at[dst_v.at[1]], sem, add=True).wait()
    plsc.subcore_barrier()

    def wback(off, nr):
        pltpu.sync_copy(deg_sh.at[pl.ds(off, nr)], part_v.at[pl.ds(0, nr)])
        pltpu.sync_copy(part_v.at[pl.ds(0, nr)],
                        deg_hbm.at[c].at[pl.ds(off, nr)])

    _row_partition(s, wback)


def _deg_call(dst, zeros_n, ones_e):
    mesh = plsc.VectorSubcoreMesh(core_axis_name="c", subcore_axis_name="s",
                                  num_cores=NC, num_subcores=NS)
    return pl.kernel(
        _deg_body,
        out_type=jax.ShapeDtypeStruct((NC, N, DW), jnp.float32),
        mesh=mesh,
        scratch_types=[
            pltpu.VMEM((2, CHUNKA), jnp.int32),
            pltpu.VMEM((CHUNKA, DW), jnp.float32),
            pltpu.VMEM((ROWS_BIG, DW), jnp.float32),
            pltpu.VMEM_SHARED((N, DW), jnp.float32),
            pltpu.SemaphoreType.DMA,
        ],
        compiler_params=pltpu.CompilerParams(use_tc_tiling_on_sc=False),
    )(dst, zeros_n, ones_e)


def _mm_body(nf_ref, w_ref, deg_ref, xs_ref, dinv_ref):
    x = jnp.dot(nf_ref[...], w_ref[...], preferred_element_type=jnp.float32)
    deg = deg_ref[0] + deg_ref[1] + 1.0
    dinv = lax.rsqrt(deg)
    xs = x * dinv
    dinv_ref[...] = dinv
    xs_ref[0] = xs[:, :HH]
    xs_ref[1] = xs[:, HH:]


CHUNK = 1000
NCHUNK = E_PER_TILE_C // CHUNK


def _agg_body(xs_hbm, src_hbm, dst_hbm, agg_hbm, sidx_v, didx_v,
              rows0, rows1, init_v, agg_sh, gsem0, gsem1, ssem0, ssem1):
    c = lax.axis_index("c")
    s = lax.axis_index("s")
    rows = [rows0, rows1]
    gsem = [gsem0, gsem1]
    ssem = [ssem0, ssem1]
    xs_c = xs_hbm.at[c]

    pltpu.sync_copy(src_hbm.at[pl.ds(s * NCHUNK, NCHUNK)], sidx_v)
    pltpu.sync_copy(dst_hbm.at[pl.ds(s * NCHUNK, NCHUNK)], didx_v)

    def xinit(off, nr):
        pltpu.sync_copy(xs_c.at[pl.ds(off, nr)], init_v.at[pl.ds(0, nr)])
        pltpu.sync_copy(init_v.at[pl.ds(0, nr)], agg_sh.at[pl.ds(off, nr)])

    _row_partition(s, xinit)

    g = [None, None]
    sc = [None, None]
    g[0] = pltpu.async_copy(xs_c.at[sidx_v.at[0]], rows[0], gsem[0])
    plsc.subcore_barrier()

    for j in range(NCHUNK):
        b = j & 1
        nb = (j + 1) & 1
        if j + 1 < NCHUNK:
            if j >= 1:
                sc[nb].wait()
            g[nb] = pltpu.async_copy(xs_c.at[sidx_v.at[j + 1]], rows[nb],
                                     gsem[nb])
        g[b].wait()
        sc[b] = pltpu.async_copy(rows[b], agg_sh.at[didx_v.at[j]], ssem[b],
                                 add=True)
    sc[(NCHUNK - 1) & 1].wait()
    sc[(NCHUNK - 2) & 1].wait()
    plsc.subcore_barrier()

    def wback(off, nr):
        pltpu.sync_copy(agg_sh.at[pl.ds(off, nr)], init_v.at[pl.ds(0, nr)])
        pltpu.sync_copy(init_v.at[pl.ds(0, nr)],
                        agg_hbm.at[c].at[pl.ds(off, nr)])

    _row_partition(s, wback)


def _agg_call(xs, src2, dst2):
    mesh = plsc.VectorSubcoreMesh(core_axis_name="c", subcore_axis_name="s",
                                  num_cores=NC, num_subcores=NS)
    return pl.kernel(
        _agg_body,
        out_type=jax.ShapeDtypeStruct((NC, N, HH), jnp.float32),
        mesh=mesh,
        scratch_types=[
            pltpu.VMEM((NCHUNK, CHUNK), jnp.int32),
            pltpu.VMEM((NCHUNK, CHUNK), jnp.int32),
            pltpu.VMEM((CHUNK, HH), jnp.float32),
            pltpu.VMEM((CHUNK, HH), jnp.float32),
            pltpu.VMEM((ROWS_BIG, HH), jnp.float32),
            pltpu.VMEM_SHARED((N, HH), jnp.float32),
            pltpu.SemaphoreType.DMA,
            pltpu.SemaphoreType.DMA,
            pltpu.SemaphoreType.DMA,
            pltpu.SemaphoreType.DMA,
        ],
        compiler_params=pltpu.CompilerParams(use_tc_tiling_on_sc=False),
    )(xs, src2, dst2)


def _pool_body(agg_ref, dinv_ref, b_ref, batch_ref, out_ref):
    agg = jnp.concatenate([agg_ref[0], agg_ref[1]], axis=1)
    dinv = dinv_ref[...]
    p = jnp.maximum(agg * dinv + b_ref[...], 0.0)
    ids = lax.broadcasted_iota(jnp.int32, (N, G), 1)
    onehot = (batch_ref[...] == ids).astype(jnp.float32)
    sums = lax.dot_general(onehot, p, (((0,), (0,)), ((), ())),
                           preferred_element_type=jnp.float32)
    counts = jnp.sum(onehot, axis=0)[:, None]
    out_ref[...] = sums / jnp.maximum(counts, 1.0)


def _pool_call(agg, dinv, b, batch):
    return pl.pallas_call(
        _pool_body,
        out_shape=jax.ShapeDtypeStruct((G, H), jnp.float32),
    )(agg, dinv, b, batch)


MM_BLK = 1000


@jax.jit
def kernel(node_feat, edge_index, batch, W, b):
    src = edge_index[0]
    dst = edge_index[1]
    zeros_n = jnp.zeros((N, DW), jnp.float32)
    ones_e = jnp.ones((CHUNKA, DW), jnp.float32)
    deg_wide = _deg_call(dst.reshape(2 * NC * NS, CHUNKA), zeros_n,
                         ones_e)
    deg_part = deg_wide[:, :, 0:1]
    xs, dinv = pl.pallas_call(
        _mm_body,
        grid=(N // MM_BLK,),
        in_specs=[
            pl.BlockSpec((MM_BLK, D), lambda i: (i, 0)),
            pl.BlockSpec((D, H), lambda i: (0, 0)),
            pl.BlockSpec((NC, MM_BLK, 1), lambda i: (0, i, 0)),
        ],
        out_specs=[
            pl.BlockSpec((NC, MM_BLK, HH), lambda i: (0, i, 0)),
            pl.BlockSpec((MM_BLK, 1), lambda i: (i, 0)),
        ],
        out_shape=[
            jax.ShapeDtypeStruct((NC, N, HH), jnp.float32),
            jax.ShapeDtypeStruct((N, 1), jnp.float32),
        ],
    )(node_feat, W, deg_part)
    agg = _agg_call(xs, src.reshape(NS * NCHUNK, CHUNK),
                    dst.reshape(NS * NCHUNK, CHUNK))
    pooled = _pool_call(agg, dinv, b.reshape(1, H), batch.reshape(N, 1))
    return pooled
